# baseline (speedup 1.0000x reference)
import functools

import jax
import jax.numpy as jnp
from jax.experimental import pallas as pl

N = 4096
IN = 256
OUT = 256
HID = 64
HEADS = 2
BI = 256

_PREC = jax.lax.Precision.HIGHEST


def _dot(a, b):
    return jax.lax.dot_general(a, b, (((1,), (0,)), ((), ())),
                               precision=_PREC,
                               preferred_element_type=jnp.float32)


def _leaky_relu(x, slope=0.2):
    return jnp.where(x >= 0, x, slope * x)


def _layer_norm(x, g, b, eps=1e-5):
    mu = jnp.mean(x, axis=-1, keepdims=True)
    var = jnp.mean((x - mu) ** 2, axis=-1, keepdims=True)
    return (x - mu) * jax.lax.rsqrt(var + eps) * g + b


def _gelu_exact(x):
    return 0.5 * x * (1.0 + jax.lax.erf(x * (2.0 ** -0.5)))


def _h1_kernel(x_ref, w_ref, o_ref):
    o_ref[:, :] = _dot(x_ref[:, :], w_ref[:, :])


def _gat1_kernel(g_ref, h_ref, ai_ref, aj_ref, b0_ref, w1_ref, h2_ref):
    pid = pl.program_id(0)
    g = g_ref[:, :]
    rows = pid * BI + jax.lax.broadcasted_iota(jnp.int32, (BI, N), 0)
    cols = jax.lax.broadcasted_iota(jnp.int32, (BI, N), 1)
    mask = (g > 0) | (cols == rows)
    mask_f = mask.astype(jnp.float32)
    neg = jnp.float32(-1e30)
    outs = []
    for hd in range(HEADS):
        hh = h_ref[:, hd * HID:(hd + 1) * HID]
        hh_blk = h_ref[pl.ds(pid * BI, BI), hd * HID:(hd + 1) * HID]
        ci = _dot(hh_blk, ai_ref[hd, :].reshape(HID, 1))
        cj = _dot(hh, aj_ref[hd, :].reshape(HID, 1))
        e = _leaky_relu(ci + cj.reshape(1, N))
        e = jnp.where(mask, e, neg)
        m = jnp.max(e, axis=-1, keepdims=True)
        p = jnp.exp(e - m) * mask_f
        s = jnp.sum(p, axis=-1, keepdims=True)
        outs.append(_dot(p, hh) / s)
    o = jnp.concatenate(outs, axis=-1) + b0_ref[0, :]
    m1 = jnp.where(o > 0, o, jnp.exp(jnp.minimum(o, 0.0)) - 1.0)
    h2_ref[:, :] = _dot(m1, w1_ref[:, :])


def _gat2_kernel(g_ref, h2_ref, x_ref, ai_ref, aj_ref, b1_ref,
                 ln1g_ref, ln1b_ref, wl_ref, bl_ref,
                 we1_ref, be1_ref, we2_ref, be2_ref,
                 ln2g_ref, ln2b_ref, wo_ref, bo_ref,
                 lnog_ref, lnob_ref, out_ref):
    pid = pl.program_id(0)
    g = g_ref[:, :]
    rows = pid * BI + jax.lax.broadcasted_iota(jnp.int32, (BI, N), 0)
    cols = jax.lax.broadcasted_iota(jnp.int32, (BI, N), 1)
    mask = (g > 0) | (cols == rows)
    mask_f = mask.astype(jnp.float32)
    neg = jnp.float32(-1e30)
    h2 = h2_ref[:, :]
    h2_blk = h2_ref[pl.ds(pid * BI, BI), :]
    ci = _dot(h2_blk, ai_ref[0, :].reshape(OUT, 1))
    cj = _dot(h2, aj_ref[0, :].reshape(OUT, 1))
    e = _leaky_relu(ci + cj.reshape(1, N))
    e = jnp.where(mask, e, neg)
    m = jnp.max(e, axis=-1, keepdims=True)
    p = jnp.exp(e - m) * mask_f
    s = jnp.sum(p, axis=-1, keepdims=True)
    o = _dot(p, h2) / s + b1_ref[0, :]

    cat = jnp.concatenate([x_ref[:, :], o], axis=-1)
    x = _layer_norm(cat, ln1g_ref[0, :], ln1b_ref[0, :])
    mm = _dot(x, wl_ref[:, :]) + bl_ref[0, :]
    enc = _dot(_gelu_exact(_dot(mm, we1_ref[:, :]) + be1_ref[0, :]),
               we2_ref[:, :]) + be2_ref[0, :]
    out = _layer_norm(mm + enc, ln2g_ref[0, :], ln2b_ref[0, :])
    out = _layer_norm(_gelu_exact(_dot(out, wo_ref[:, :]) + bo_ref[0, :]),
                      lnog_ref[0, :], lnob_ref[0, :])
    out_ref[:, :] = out


def _full(shape):
    nd = len(shape)
    return pl.BlockSpec(shape, lambda i: (0,) * nd)


@functools.partial(jax.jit, static_argnames=())
def kernel(input, graph, W0, ai0, aj0, b0, W1, ai1, aj1, b1, ln1_g, ln1_b,
           Wl, bl, We1, be1, We2, be2, ln2_g, ln2_b, Wo, bo, lno_g, lno_b):
    nblk = N // BI
    row2 = lambda v: v.reshape(1, -1)

    h1 = pl.pallas_call(
        _h1_kernel,
        grid=(nblk,),
        in_specs=[pl.BlockSpec((BI, IN), lambda i: (i, 0)), _full((IN, HEADS * HID))],
        out_specs=pl.BlockSpec((BI, HEADS * HID), lambda i: (i, 0)),
        out_shape=jax.ShapeDtypeStruct((N, HEADS * HID), jnp.float32),
    )(input, W0)

    h2 = pl.pallas_call(
        _gat1_kernel,
        grid=(nblk,),
        in_specs=[
            pl.BlockSpec((BI, N), lambda i: (i, 0)),
            _full((N, HEADS * HID)),
            _full((HEADS, HID)),
            _full((HEADS, HID)),
            _full((1, HEADS * HID)),
            _full((HEADS * HID, OUT)),
        ],
        out_specs=pl.BlockSpec((BI, OUT), lambda i: (i, 0)),
        out_shape=jax.ShapeDtypeStruct((N, OUT), jnp.float32),
    )(graph, h1, ai0, aj0, row2(b0), W1)

    out = pl.pallas_call(
        _gat2_kernel,
        grid=(nblk,),
        in_specs=[
            pl.BlockSpec((BI, N), lambda i: (i, 0)),
            _full((N, OUT)),
            pl.BlockSpec((BI, IN), lambda i: (i, 0)),
            _full((1, OUT)),
            _full((1, OUT)),
            _full((1, OUT)),
            _full((1, IN + OUT)),
            _full((1, IN + OUT)),
            _full((IN + OUT, OUT)),
            _full((1, OUT)),
            _full((OUT, OUT)),
            _full((1, OUT)),
            _full((OUT, OUT)),
            _full((1, OUT)),
            _full((1, OUT)),
            _full((1, OUT)),
            _full((OUT, OUT)),
            _full((1, OUT)),
            _full((1, OUT)),
            _full((1, OUT)),
        ],
        out_specs=pl.BlockSpec((BI, OUT), lambda i: (i, 0)),
        out_shape=jax.ShapeDtypeStruct((N, OUT), jnp.float32),
    )(graph, h2, input, ai1, aj1, row2(b1), row2(ln1_g), row2(ln1_b), Wl,
      row2(bl), We1, row2(be1), We2, row2(be2), row2(ln2_g), row2(ln2_b),
      Wo, row2(bo), row2(lno_g), row2(lno_b))
    return out


# speedup vs baseline: 3.4751x; 3.4751x over previous
import jax
import jax.numpy as jnp
from jax.experimental import pallas as pl

N = 4096
IN = 256
OUT = 256
HID = 64
HEADS = 2
BI = 256

_BIG = 1e30


def _dot(a, b):
    return jnp.dot(a, b, preferred_element_type=jnp.float32)


def _leaky(x, slope=0.2):
    return jnp.maximum(x, slope * x)


def _layer_norm(x, g, b, eps=1e-5):
    mu = jnp.mean(x, axis=-1, keepdims=True)
    var = jnp.mean((x - mu) ** 2, axis=-1, keepdims=True)
    return (x - mu) * jax.lax.rsqrt(var + eps) * g + b


def _gelu_exact(x):
    return 0.5 * x * (1.0 + jax.lax.erf(x * (2.0 ** -0.5)))


def _log_mask(g_ref):
    gf = g_ref[:, :].astype(jnp.float32)
    return gf * _BIG - _BIG


def _self_loop_weight(g_ref, pid):
    gsub = g_ref[:, pl.ds(pid * BI, BI)].astype(jnp.float32)
    r = jax.lax.broadcasted_iota(jnp.int32, (BI, BI), 0)
    c = jax.lax.broadcasted_iota(jnp.int32, (BI, BI), 1)
    gdiag = jnp.sum(jnp.where(r == c, gsub, 0.0), axis=1, keepdims=True)
    return 1.0 - gdiag


def _h1_kernel(x_ref, w_ref, o_ref):
    o_ref[:, :] = _dot(x_ref[:, :], w_ref[:, :])


def _gat1_kernel(g_ref, h_ref, ai_ref, aj_ref, b0_ref, w1_ref, h2_ref):
    pid = pl.program_id(0)
    logm = _log_mask(g_ref)
    w_self = _self_loop_weight(g_ref, pid)

    cis, cjs, cjbs, hblks = [], [], [], []
    for hd in range(HEADS):
        hh_blk = h_ref[pl.ds(pid * BI, BI), hd * HID:(hd + 1) * HID]
        hblks.append(hh_blk)
        cis.append(_dot(hh_blk, ai_ref[hd, :].reshape(HID, 1)))
        cjs.append(_dot(h_ref[:, hd * HID:(hd + 1) * HID],
                        aj_ref[hd, :].reshape(HID, 1)))
        cjbs.append(_dot(hh_blk, aj_ref[hd, :].reshape(HID, 1)))
    shift = _leaky(jnp.maximum(
        jnp.max(cis[0]) + jnp.max(cjs[0]),
        jnp.max(cis[1]) + jnp.max(cjs[1])))
    mm = logm - shift

    outs = []
    for hd in range(HEADS):
        z = _leaky(cis[hd] + cjs[hd].reshape(1, N))
        p = jnp.exp(z + mm)
        s = jnp.sum(p, axis=-1, keepdims=True)
        pd = w_self * jnp.exp(_leaky(cis[hd] + cjbs[hd]) - shift)
        hh = h_ref[:, hd * HID:(hd + 1) * HID]
        outs.append((_dot(p, hh) + pd * hblks[hd]) / (s + pd))
    o = jnp.concatenate(outs, axis=-1) + b0_ref[0, :]
    m1 = jnp.where(o > 0, o, jnp.exp(jnp.minimum(o, 0.0)) - 1.0)
    h2_ref[:, :] = _dot(m1, w1_ref[:, :])


def _gat2_kernel(g_ref, h2_ref, x_ref, ai_ref, aj_ref, b1_ref,
                 ln1g_ref, ln1b_ref, wl_ref, bl_ref,
                 we1_ref, be1_ref, we2_ref, be2_ref,
                 ln2g_ref, ln2b_ref, wo_ref, bo_ref,
                 lnog_ref, lnob_ref, out_ref):
    pid = pl.program_id(0)
    logm = _log_mask(g_ref)
    w_self = _self_loop_weight(g_ref, pid)

    h2 = h2_ref[:, :]
    h2_blk = h2_ref[pl.ds(pid * BI, BI), :]
    ci = _dot(h2_blk, ai_ref[0, :].reshape(OUT, 1))
    cj = _dot(h2, aj_ref[0, :].reshape(OUT, 1))
    cjb = _dot(h2_blk, aj_ref[0, :].reshape(OUT, 1))
    shift = _leaky(jnp.max(ci) + jnp.max(cj))
    z = _leaky(ci + cj.reshape(1, N))
    p = jnp.exp(z + (logm - shift))
    s = jnp.sum(p, axis=-1, keepdims=True)
    pd = w_self * jnp.exp(_leaky(ci + cjb) - shift)
    o = (_dot(p, h2) + pd * h2_blk) / (s + pd) + b1_ref[0, :]

    cat = jnp.concatenate([x_ref[:, :], o], axis=-1)
    x = _layer_norm(cat, ln1g_ref[0, :], ln1b_ref[0, :])
    mm = _dot(x, wl_ref[:, :]) + bl_ref[0, :]
    enc = _dot(_gelu_exact(_dot(mm, we1_ref[:, :]) + be1_ref[0, :]),
               we2_ref[:, :]) + be2_ref[0, :]
    out = _layer_norm(mm + enc, ln2g_ref[0, :], ln2b_ref[0, :])
    out = _layer_norm(_gelu_exact(_dot(out, wo_ref[:, :]) + bo_ref[0, :]),
                      lnog_ref[0, :], lnob_ref[0, :])
    out_ref[:, :] = out


def _full(shape):
    nd = len(shape)
    return pl.BlockSpec(shape, lambda i: (0,) * nd)


def kernel(input, graph, W0, ai0, aj0, b0, W1, ai1, aj1, b1, ln1_g, ln1_b,
           Wl, bl, We1, be1, We2, be2, ln2_g, ln2_b, Wo, bo, lno_g, lno_b):
    nblk = N // BI
    row2 = lambda v: v.reshape(1, -1)

    h1 = pl.pallas_call(
        _h1_kernel,
        grid=(nblk,),
        in_specs=[pl.BlockSpec((BI, IN), lambda i: (i, 0)),
                  _full((IN, HEADS * HID))],
        out_specs=pl.BlockSpec((BI, HEADS * HID), lambda i: (i, 0)),
        out_shape=jax.ShapeDtypeStruct((N, HEADS * HID), jnp.float32),
    )(input, W0)

    h2 = pl.pallas_call(
        _gat1_kernel,
        grid=(nblk,),
        in_specs=[
            pl.BlockSpec((BI, N), lambda i: (i, 0)),
            _full((N, HEADS * HID)),
            _full((HEADS, HID)),
            _full((HEADS, HID)),
            _full((1, HEADS * HID)),
            _full((HEADS * HID, OUT)),
        ],
        out_specs=pl.BlockSpec((BI, OUT), lambda i: (i, 0)),
        out_shape=jax.ShapeDtypeStruct((N, OUT), jnp.float32),
    )(graph, h1, ai0, aj0, row2(b0), W1)

    out = pl.pallas_call(
        _gat2_kernel,
        grid=(nblk,),
        in_specs=[
            pl.BlockSpec((BI, N), lambda i: (i, 0)),
            _full((N, OUT)),
            pl.BlockSpec((BI, IN), lambda i: (i, 0)),
            _full((1, OUT)),
            _full((1, OUT)),
            _full((1, OUT)),
            _full((1, IN + OUT)),
            _full((1, IN + OUT)),
            _full((IN + OUT, OUT)),
            _full((1, OUT)),
            _full((OUT, OUT)),
            _full((1, OUT)),
            _full((OUT, OUT)),
            _full((1, OUT)),
            _full((1, OUT)),
            _full((1, OUT)),
            _full((OUT, OUT)),
            _full((1, OUT)),
            _full((1, OUT)),
            _full((1, OUT)),
        ],
        out_specs=pl.BlockSpec((BI, OUT), lambda i: (i, 0)),
        out_shape=jax.ShapeDtypeStruct((N, OUT), jnp.float32),
    )(graph, h2, input, ai1, aj1, row2(b1), row2(ln1_g), row2(ln1_b), Wl,
      row2(bl), We1, row2(be1), We2, row2(be2), row2(ln2_g), row2(ln2_b),
      Wo, row2(bo), row2(lno_g), row2(lno_b))
    return out


# speedup vs baseline: 3.9524x; 1.1373x over previous
import jax
import jax.numpy as jnp
from jax.experimental import pallas as pl

N = 4096
IN = 256
OUT = 256
HID = 64
HEADS = 2
BI = 256
HW = 128
H2W = 384


def _dot(a, b):
    return jnp.dot(a, b, preferred_element_type=jnp.float32)


def _leaky(x, slope=0.2):
    return jnp.maximum(x, slope * x)


def _layer_norm(x, g, b, eps=1e-5):
    mu = jnp.mean(x, axis=-1, keepdims=True)
    var = jnp.mean((x - mu) ** 2, axis=-1, keepdims=True)
    return (x - mu) * jax.lax.rsqrt(var + eps) * g + b


def _gelu_exact(x):
    return 0.5 * x * (1.0 + jax.lax.erf(x * (2.0 ** -0.5)))


def _self_loop_weight(g_ref, pid):
    gsub = g_ref[:, pl.ds(pid * BI, BI)].astype(jnp.float32)
    r = jax.lax.broadcasted_iota(jnp.int32, (BI, BI), 0)
    c = jax.lax.broadcasted_iota(jnp.int32, (BI, BI), 1)
    gdiag = jnp.sum(jnp.where(r == c, gsub, 0.0), axis=1, keepdims=True)
    return 1.0 - gdiag


def _h1_kernel(x_ref, w_ref, o_ref):
    h = _dot(x_ref[:, :], w_ref[:, :])
    ones = jnp.ones((h.shape[0], 1), jnp.float32)
    zeros = jnp.zeros((h.shape[0], HW - HID - 1), jnp.float32)
    o_ref[:, :] = jnp.concatenate(
        [h[:, :HID], ones, zeros, h[:, HID:], ones, zeros], axis=-1)


def _gat1_kernel(g_ref, h_ref, ai_ref, aj_ref, b0_ref, w1_ref, h2_ref):
    pid = pl.program_id(0)
    gf = g_ref[:, :].astype(jnp.float32)
    w_self = _self_loop_weight(g_ref, pid)

    cis, cjs, cjbs, hblks = [], [], [], []
    for hd in range(HEADS):
        hh_blk = h_ref[pl.ds(pid * BI, BI), hd * HW:hd * HW + HID]
        hblks.append(h_ref[pl.ds(pid * BI, BI), hd * HW:(hd + 1) * HW])
        cis.append(_dot(hh_blk, ai_ref[hd, :].reshape(HID, 1)))
        cjs.append(_dot(h_ref[:, hd * HW:hd * HW + HID],
                        aj_ref[hd, :].reshape(HID, 1)))
        cjbs.append(_dot(hh_blk, aj_ref[hd, :].reshape(HID, 1)))
    shift = _leaky(jnp.maximum(
        jnp.max(cis[0]) + jnp.max(cjs[0]),
        jnp.max(cis[1]) + jnp.max(cjs[1])))

    outs = []
    for hd in range(HEADS):
        ci_s = cis[hd] - shift
        ci2_s = 0.2 * cis[hd] - shift
        cj = cjs[hd].reshape(1, N)
        cj2 = 0.2 * cj
        z = jnp.maximum(ci_s + cj, ci2_s + cj2)
        p = jnp.exp(z) * gf
        pd = w_self * jnp.exp(
            _leaky(cis[hd] + cjbs[hd]) - shift)
        o_aug = _dot(p, h_ref[:, hd * HW:(hd + 1) * HW]) + pd * hblks[hd]
        outs.append(o_aug[:, :HID] / o_aug[:, HID:HID + 1])
    o = jnp.concatenate(outs, axis=-1) + b0_ref[0, :]
    m1 = jnp.where(o > 0, o, jnp.exp(jnp.minimum(o, 0.0)) - 1.0)
    h2 = _dot(m1, w1_ref[:, :])
    ones = jnp.ones((BI, 1), jnp.float32)
    zeros = jnp.zeros((BI, H2W - OUT - 1), jnp.float32)
    h2_ref[:, :] = jnp.concatenate([h2, ones, zeros], axis=-1)


def _gat2_kernel(g_ref, h2_ref, x_ref, ai_ref, aj_ref, b1_ref,
                 ln1g_ref, ln1b_ref, wl_ref, bl_ref,
                 we1_ref, be1_ref, we2_ref, be2_ref,
                 ln2g_ref, ln2b_ref, wo_ref, bo_ref,
                 lnog_ref, lnob_ref, out_ref):
    pid = pl.program_id(0)
    gf = g_ref[:, :].astype(jnp.float32)
    w_self = _self_loop_weight(g_ref, pid)

    h2_blk = h2_ref[pl.ds(pid * BI, BI), 0:OUT]
    h2aug_blk = h2_ref[pl.ds(pid * BI, BI), :]
    ci = _dot(h2_blk, ai_ref[0, :].reshape(OUT, 1))
    cj = _dot(h2_ref[:, 0:OUT], aj_ref[0, :].reshape(OUT, 1))
    cjb = _dot(h2_blk, aj_ref[0, :].reshape(OUT, 1))
    shift = _leaky(jnp.max(ci) + jnp.max(cj))

    ci_s = ci - shift
    ci2_s = 0.2 * ci - shift
    cjt = cj.reshape(1, N)
    z = jnp.maximum(ci_s + cjt, ci2_s + 0.2 * cjt)
    p = jnp.exp(z) * gf
    pd = w_self * jnp.exp(_leaky(ci + cjb) - shift)
    o_aug = _dot(p, h2_ref[:, :]) + pd * h2aug_blk
    o = o_aug[:, :OUT] / o_aug[:, OUT:OUT + 1] + b1_ref[0, :]

    cat = jnp.concatenate([x_ref[:, :], o], axis=-1)
    x = _layer_norm(cat, ln1g_ref[0, :], ln1b_ref[0, :])
    mm = _dot(x, wl_ref[:, :]) + bl_ref[0, :]
    enc = _dot(_gelu_exact(_dot(mm, we1_ref[:, :]) + be1_ref[0, :]),
               we2_ref[:, :]) + be2_ref[0, :]
    out = _layer_norm(mm + enc, ln2g_ref[0, :], ln2b_ref[0, :])
    out = _layer_norm(_gelu_exact(_dot(out, wo_ref[:, :]) + bo_ref[0, :]),
                      lnog_ref[0, :], lnob_ref[0, :])
    out_ref[:, :] = out


def _full(shape):
    nd = len(shape)
    return pl.BlockSpec(shape, lambda i: (0,) * nd)


def kernel(input, graph, W0, ai0, aj0, b0, W1, ai1, aj1, b1, ln1_g, ln1_b,
           Wl, bl, We1, be1, We2, be2, ln2_g, ln2_b, Wo, bo, lno_g, lno_b):
    nblk = N // BI
    row2 = lambda v: v.reshape(1, -1)

    h1 = pl.pallas_call(
        _h1_kernel,
        grid=(nblk,),
        in_specs=[pl.BlockSpec((BI, IN), lambda i: (i, 0)),
                  _full((IN, HEADS * HID))],
        out_specs=pl.BlockSpec((BI, HEADS * HW), lambda i: (i, 0)),
        out_shape=jax.ShapeDtypeStruct((N, HEADS * HW), jnp.float32),
    )(input, W0)

    h2 = pl.pallas_call(
        _gat1_kernel,
        grid=(nblk,),
        in_specs=[
            pl.BlockSpec((BI, N), lambda i: (i, 0)),
            _full((N, HEADS * HW)),
            _full((HEADS, HID)),
            _full((HEADS, HID)),
            _full((1, HEADS * HID)),
            _full((HEADS * HID, OUT)),
        ],
        out_specs=pl.BlockSpec((BI, H2W), lambda i: (i, 0)),
        out_shape=jax.ShapeDtypeStruct((N, H2W), jnp.float32),
    )(graph, h1, ai0, aj0, row2(b0), W1)

    out = pl.pallas_call(
        _gat2_kernel,
        grid=(nblk,),
        in_specs=[
            pl.BlockSpec((BI, N), lambda i: (i, 0)),
            _full((N, H2W)),
            pl.BlockSpec((BI, IN), lambda i: (i, 0)),
            _full((1, OUT)),
            _full((1, OUT)),
            _full((1, OUT)),
            _full((1, IN + OUT)),
            _full((1, IN + OUT)),
            _full((IN + OUT, OUT)),
            _full((1, OUT)),
            _full((OUT, OUT)),
            _full((1, OUT)),
            _full((OUT, OUT)),
            _full((1, OUT)),
            _full((1, OUT)),
            _full((1, OUT)),
            _full((OUT, OUT)),
            _full((1, OUT)),
            _full((1, OUT)),
            _full((1, OUT)),
        ],
        out_specs=pl.BlockSpec((BI, OUT), lambda i: (i, 0)),
        out_shape=jax.ShapeDtypeStruct((N, OUT), jnp.float32),
    )(graph, h2, input, ai1, aj1, row2(b1), row2(ln1_g), row2(ln1_b), Wl,
      row2(bl), We1, row2(be1), We2, row2(be2), row2(ln2_g), row2(ln2_b),
      Wo, row2(bo), row2(lno_g), row2(lno_b))
    return out


# speedup vs baseline: 5.2146x; 1.3194x over previous
import jax
import jax.numpy as jnp
from jax.experimental import pallas as pl

N = 4096
IN = 256
OUT = 256
HID = 64
HEADS = 2
BI = 512
HW = 128
H2W = 384


def _dot(a, b):
    return jnp.dot(a, b, preferred_element_type=jnp.float32)


def _leaky(x, slope=0.2):
    return jnp.maximum(x, slope * x)


def _layer_norm(x, g, b, eps=1e-5):
    mu = jnp.mean(x, axis=-1, keepdims=True)
    var = jnp.mean((x - mu) ** 2, axis=-1, keepdims=True)
    return (x - mu) * jax.lax.rsqrt(var + eps) * g + b


def _gelu_exact(x):
    return 0.5 * x * (1.0 + jax.lax.erf(x * (2.0 ** -0.5)))


def _self_loop_weight(g_ref, pid):
    gsub = g_ref[:, pl.ds(pid * BI, BI)].astype(jnp.float32)
    r = jax.lax.broadcasted_iota(jnp.int32, (BI, BI), 0)
    c = jax.lax.broadcasted_iota(jnp.int32, (BI, BI), 1)
    gdiag = jnp.sum(jnp.where(r == c, gsub, 0.0), axis=1, keepdims=True)
    return 1.0 - gdiag


def _h1_kernel(x_ref, w_ref, o_ref):
    h = _dot(x_ref[:, :], w_ref[:, :])
    ones = jnp.ones((h.shape[0], 1), jnp.float32)
    zeros = jnp.zeros((h.shape[0], HW - HID - 1), jnp.float32)
    o_ref[:, :] = jnp.concatenate(
        [h[:, :HID], ones, zeros, h[:, HID:], ones, zeros], axis=-1)


def _gat1_kernel(g_ref, h_ref, ai_ref, aj_ref, b0_ref, w1_ref, h2_ref):
    pid = pl.program_id(0)
    gf = g_ref[:, :].astype(jnp.float32)
    w_self = _self_loop_weight(g_ref, pid)

    cis, cjs, cjbs, hblks = [], [], [], []
    for hd in range(HEADS):
        hh_blk = h_ref[pl.ds(pid * BI, BI), hd * HW:hd * HW + HID]
        hblks.append(h_ref[pl.ds(pid * BI, BI), hd * HW:(hd + 1) * HW])
        cis.append(_dot(hh_blk, ai_ref[hd, :].reshape(HID, 1)))
        cjs.append(_dot(h_ref[:, hd * HW:hd * HW + HID],
                        aj_ref[hd, :].reshape(HID, 1)))
        cjbs.append(_dot(hh_blk, aj_ref[hd, :].reshape(HID, 1)))
    shift = _leaky(jnp.maximum(
        jnp.max(cis[0]) + jnp.max(cjs[0]),
        jnp.max(cis[1]) + jnp.max(cjs[1])))

    outs = []
    for hd in range(HEADS):
        ci_s = cis[hd] - shift
        ci2_s = 0.2 * cis[hd] - shift
        cj = cjs[hd].reshape(1, N)
        cj2 = 0.2 * cj
        z = jnp.maximum(ci_s + cj, ci2_s + cj2)
        p = jnp.exp(z) * gf
        pd = w_self * jnp.exp(
            _leaky(cis[hd] + cjbs[hd]) - shift)
        o_aug = _dot(p, h_ref[:, hd * HW:(hd + 1) * HW]) + pd * hblks[hd]
        outs.append(o_aug[:, :HID] / o_aug[:, HID:HID + 1])
    o = jnp.concatenate(outs, axis=-1) + b0_ref[0, :]
    m1 = jnp.where(o > 0, o, jnp.exp(jnp.minimum(o, 0.0)) - 1.0)
    h2 = _dot(m1, w1_ref[:, :])
    ones = jnp.ones((BI, 1), jnp.float32)
    zeros = jnp.zeros((BI, H2W - OUT - 1), jnp.float32)
    h2_ref[:, :] = jnp.concatenate([h2, ones, zeros], axis=-1)


def _gat2_kernel(g_ref, h2_ref, x_ref, ai_ref, aj_ref, b1_ref,
                 ln1g_ref, ln1b_ref, wl_ref, bl_ref,
                 we1_ref, be1_ref, we2_ref, be2_ref,
                 ln2g_ref, ln2b_ref, wo_ref, bo_ref,
                 lnog_ref, lnob_ref, out_ref):
    pid = pl.program_id(0)
    gf = g_ref[:, :].astype(jnp.float32)
    w_self = _self_loop_weight(g_ref, pid)

    h2_blk = h2_ref[pl.ds(pid * BI, BI), 0:OUT]
    h2aug_blk = h2_ref[pl.ds(pid * BI, BI), :]
    ci = _dot(h2_blk, ai_ref[0, :].reshape(OUT, 1))
    cj = _dot(h2_ref[:, 0:OUT], aj_ref[0, :].reshape(OUT, 1))
    cjb = _dot(h2_blk, aj_ref[0, :].reshape(OUT, 1))
    shift = _leaky(jnp.max(ci) + jnp.max(cj))

    ci_s = ci - shift
    ci2_s = 0.2 * ci - shift
    cjt = cj.reshape(1, N)
    z = jnp.maximum(ci_s + cjt, ci2_s + 0.2 * cjt)
    p = jnp.exp(z) * gf
    pd = w_self * jnp.exp(_leaky(ci + cjb) - shift)
    o_aug = _dot(p, h2_ref[:, :]) + pd * h2aug_blk
    o = o_aug[:, :OUT] / o_aug[:, OUT:OUT + 1] + b1_ref[0, :]

    cat = jnp.concatenate([x_ref[:, :], o], axis=-1)
    x = _layer_norm(cat, ln1g_ref[0, :], ln1b_ref[0, :])
    mm = _dot(x, wl_ref[:, :]) + bl_ref[0, :]
    enc = _dot(_gelu_exact(_dot(mm, we1_ref[:, :]) + be1_ref[0, :]),
               we2_ref[:, :]) + be2_ref[0, :]
    out = _layer_norm(mm + enc, ln2g_ref[0, :], ln2b_ref[0, :])
    out = _layer_norm(_gelu_exact(_dot(out, wo_ref[:, :]) + bo_ref[0, :]),
                      lnog_ref[0, :], lnob_ref[0, :])
    out_ref[:, :] = out


def _full(shape):
    nd = len(shape)
    return pl.BlockSpec(shape, lambda i: (0,) * nd)


def kernel(input, graph, W0, ai0, aj0, b0, W1, ai1, aj1, b1, ln1_g, ln1_b,
           Wl, bl, We1, be1, We2, be2, ln2_g, ln2_b, Wo, bo, lno_g, lno_b):
    nblk = N // BI
    row2 = lambda v: v.reshape(1, -1)

    h1 = pl.pallas_call(
        _h1_kernel,
        grid=(nblk,),
        in_specs=[pl.BlockSpec((BI, IN), lambda i: (i, 0)),
                  _full((IN, HEADS * HID))],
        out_specs=pl.BlockSpec((BI, HEADS * HW), lambda i: (i, 0)),
        out_shape=jax.ShapeDtypeStruct((N, HEADS * HW), jnp.float32),
    )(input, W0)

    h2 = pl.pallas_call(
        _gat1_kernel,
        grid=(nblk,),
        in_specs=[
            pl.BlockSpec((BI, N), lambda i: (i, 0)),
            _full((N, HEADS * HW)),
            _full((HEADS, HID)),
            _full((HEADS, HID)),
            _full((1, HEADS * HID)),
            _full((HEADS * HID, OUT)),
        ],
        out_specs=pl.BlockSpec((BI, H2W), lambda i: (i, 0)),
        out_shape=jax.ShapeDtypeStruct((N, H2W), jnp.float32),
    )(graph, h1, ai0, aj0, row2(b0), W1)

    out = pl.pallas_call(
        _gat2_kernel,
        grid=(nblk,),
        in_specs=[
            pl.BlockSpec((BI, N), lambda i: (i, 0)),
            _full((N, H2W)),
            pl.BlockSpec((BI, IN), lambda i: (i, 0)),
            _full((1, OUT)),
            _full((1, OUT)),
            _full((1, OUT)),
            _full((1, IN + OUT)),
            _full((1, IN + OUT)),
            _full((IN + OUT, OUT)),
            _full((1, OUT)),
            _full((OUT, OUT)),
            _full((1, OUT)),
            _full((OUT, OUT)),
            _full((1, OUT)),
            _full((1, OUT)),
            _full((1, OUT)),
            _full((OUT, OUT)),
            _full((1, OUT)),
            _full((1, OUT)),
            _full((1, OUT)),
        ],
        out_specs=pl.BlockSpec((BI, OUT), lambda i: (i, 0)),
        out_shape=jax.ShapeDtypeStruct((N, OUT), jnp.float32),
    )(graph, h2, input, ai1, aj1, row2(b1), row2(ln1_g), row2(ln1_b), Wl,
      row2(bl), We1, row2(be1), We2, row2(be2), row2(ln2_g), row2(ln2_b),
      Wo, row2(bo), row2(lno_g), row2(lno_b))
    return out


# speedup vs baseline: 5.7934x; 1.1110x over previous
import jax
import jax.numpy as jnp
from jax.experimental import pallas as pl

N = 4096
IN = 256
OUT = 256
HID = 64
HEADS = 2
BI = 1024
HW = 128
H2W = 384


def _dot(a, b):
    return jnp.dot(a, b, preferred_element_type=jnp.float32)


def _leaky(x, slope=0.2):
    return jnp.maximum(x, slope * x)


def _layer_norm(x, g, b, eps=1e-5):
    mu = jnp.mean(x, axis=-1, keepdims=True)
    var = jnp.mean((x - mu) ** 2, axis=-1, keepdims=True)
    return (x - mu) * jax.lax.rsqrt(var + eps) * g + b


def _gelu_exact(x):
    return 0.5 * x * (1.0 + jax.lax.erf(x * (2.0 ** -0.5)))


def _self_loop_weight(g_ref, pid):
    gsub = g_ref[:, pl.ds(pid * BI, BI)].astype(jnp.float32)
    r = jax.lax.broadcasted_iota(jnp.int32, (BI, BI), 0)
    c = jax.lax.broadcasted_iota(jnp.int32, (BI, BI), 1)
    gdiag = jnp.sum(jnp.where(r == c, gsub, 0.0), axis=1, keepdims=True)
    return 1.0 - gdiag


def _h1_kernel(x_ref, w_ref, o_ref):
    h = _dot(x_ref[:, :], w_ref[:, :])
    ones = jnp.ones((h.shape[0], 1), jnp.float32)
    zeros = jnp.zeros((h.shape[0], HW - HID - 1), jnp.float32)
    o_ref[:, :] = jnp.concatenate(
        [h[:, :HID], ones, zeros, h[:, HID:], ones, zeros], axis=-1)


def _gat1_kernel(g_ref, h_ref, ai_ref, aj_ref, b0_ref, w1_ref, h2_ref):
    pid = pl.program_id(0)
    gf = g_ref[:, :].astype(jnp.float32)
    w_self = _self_loop_weight(g_ref, pid)

    cis, cjs, cjbs, hblks = [], [], [], []
    for hd in range(HEADS):
        hh_blk = h_ref[pl.ds(pid * BI, BI), hd * HW:hd * HW + HID]
        hblks.append(h_ref[pl.ds(pid * BI, BI), hd * HW:(hd + 1) * HW])
        cis.append(_dot(hh_blk, ai_ref[hd, :].reshape(HID, 1)))
        cjs.append(_dot(h_ref[:, hd * HW:hd * HW + HID],
                        aj_ref[hd, :].reshape(HID, 1)))
        cjbs.append(_dot(hh_blk, aj_ref[hd, :].reshape(HID, 1)))
    shift = _leaky(jnp.maximum(
        jnp.max(cis[0]) + jnp.max(cjs[0]),
        jnp.max(cis[1]) + jnp.max(cjs[1])))

    outs = []
    for hd in range(HEADS):
        ci_s = cis[hd] - shift
        ci2_s = 0.2 * cis[hd] - shift
        cj = cjs[hd].reshape(1, N)
        cj2 = 0.2 * cj
        z = jnp.maximum(ci_s + cj, ci2_s + cj2)
        p = jnp.exp(z) * gf
        pd = w_self * jnp.exp(
            _leaky(cis[hd] + cjbs[hd]) - shift)
        o_aug = _dot(p, h_ref[:, hd * HW:(hd + 1) * HW]) + pd * hblks[hd]
        outs.append(o_aug[:, :HID] / o_aug[:, HID:HID + 1])
    o = jnp.concatenate(outs, axis=-1) + b0_ref[0, :]
    m1 = jnp.where(o > 0, o, jnp.exp(jnp.minimum(o, 0.0)) - 1.0)
    h2 = _dot(m1, w1_ref[:, :])
    ones = jnp.ones((BI, 1), jnp.float32)
    zeros = jnp.zeros((BI, H2W - OUT - 1), jnp.float32)
    h2_ref[:, :] = jnp.concatenate([h2, ones, zeros], axis=-1)


def _gat2_kernel(g_ref, h2_ref, x_ref, ai_ref, aj_ref, b1_ref,
                 ln1g_ref, ln1b_ref, wl_ref, bl_ref,
                 we1_ref, be1_ref, we2_ref, be2_ref,
                 ln2g_ref, ln2b_ref, wo_ref, bo_ref,
                 lnog_ref, lnob_ref, out_ref):
    pid = pl.program_id(0)
    gf = g_ref[:, :].astype(jnp.float32)
    w_self = _self_loop_weight(g_ref, pid)

    h2_blk = h2_ref[pl.ds(pid * BI, BI), 0:OUT]
    h2aug_blk = h2_ref[pl.ds(pid * BI, BI), :]
    ci = _dot(h2_blk, ai_ref[0, :].reshape(OUT, 1))
    cj = _dot(h2_ref[:, 0:OUT], aj_ref[0, :].reshape(OUT, 1))
    cjb = _dot(h2_blk, aj_ref[0, :].reshape(OUT, 1))
    shift = _leaky(jnp.max(ci) + jnp.max(cj))

    ci_s = ci - shift
    ci2_s = 0.2 * ci - shift
    cjt = cj.reshape(1, N)
    z = jnp.maximum(ci_s + cjt, ci2_s + 0.2 * cjt)
    p = jnp.exp(z) * gf
    pd = w_self * jnp.exp(_leaky(ci + cjb) - shift)
    o_aug = _dot(p, h2_ref[:, :]) + pd * h2aug_blk
    o = o_aug[:, :OUT] / o_aug[:, OUT:OUT + 1] + b1_ref[0, :]

    cat = jnp.concatenate([x_ref[:, :], o], axis=-1)
    x = _layer_norm(cat, ln1g_ref[0, :], ln1b_ref[0, :])
    mm = _dot(x, wl_ref[:, :]) + bl_ref[0, :]
    enc = _dot(_gelu_exact(_dot(mm, we1_ref[:, :]) + be1_ref[0, :]),
               we2_ref[:, :]) + be2_ref[0, :]
    out = _layer_norm(mm + enc, ln2g_ref[0, :], ln2b_ref[0, :])
    out = _layer_norm(_gelu_exact(_dot(out, wo_ref[:, :]) + bo_ref[0, :]),
                      lnog_ref[0, :], lnob_ref[0, :])
    out_ref[:, :] = out


def _full(shape):
    nd = len(shape)
    return pl.BlockSpec(shape, lambda i: (0,) * nd)


def kernel(input, graph, W0, ai0, aj0, b0, W1, ai1, aj1, b1, ln1_g, ln1_b,
           Wl, bl, We1, be1, We2, be2, ln2_g, ln2_b, Wo, bo, lno_g, lno_b):
    nblk = N // BI
    row2 = lambda v: v.reshape(1, -1)

    h1 = pl.pallas_call(
        _h1_kernel,
        grid=(nblk,),
        in_specs=[pl.BlockSpec((BI, IN), lambda i: (i, 0)),
                  _full((IN, HEADS * HID))],
        out_specs=pl.BlockSpec((BI, HEADS * HW), lambda i: (i, 0)),
        out_shape=jax.ShapeDtypeStruct((N, HEADS * HW), jnp.float32),
    )(input, W0)

    h2 = pl.pallas_call(
        _gat1_kernel,
        grid=(nblk,),
        in_specs=[
            pl.BlockSpec((BI, N), lambda i: (i, 0)),
            _full((N, HEADS * HW)),
            _full((HEADS, HID)),
            _full((HEADS, HID)),
            _full((1, HEADS * HID)),
            _full((HEADS * HID, OUT)),
        ],
        out_specs=pl.BlockSpec((BI, H2W), lambda i: (i, 0)),
        out_shape=jax.ShapeDtypeStruct((N, H2W), jnp.float32),
    )(graph, h1, ai0, aj0, row2(b0), W1)

    out = pl.pallas_call(
        _gat2_kernel,
        grid=(nblk,),
        in_specs=[
            pl.BlockSpec((BI, N), lambda i: (i, 0)),
            _full((N, H2W)),
            pl.BlockSpec((BI, IN), lambda i: (i, 0)),
            _full((1, OUT)),
            _full((1, OUT)),
            _full((1, OUT)),
            _full((1, IN + OUT)),
            _full((1, IN + OUT)),
            _full((IN + OUT, OUT)),
            _full((1, OUT)),
            _full((OUT, OUT)),
            _full((1, OUT)),
            _full((OUT, OUT)),
            _full((1, OUT)),
            _full((1, OUT)),
            _full((1, OUT)),
            _full((OUT, OUT)),
            _full((1, OUT)),
            _full((1, OUT)),
            _full((1, OUT)),
        ],
        out_specs=pl.BlockSpec((BI, OUT), lambda i: (i, 0)),
        out_shape=jax.ShapeDtypeStruct((N, OUT), jnp.float32),
    )(graph, h2, input, ai1, aj1, row2(b1), row2(ln1_g), row2(ln1_b), Wl,
      row2(bl), We1, row2(be1), We2, row2(be2), row2(ln2_g), row2(ln2_b),
      Wo, row2(bo), row2(lno_g), row2(lno_b))
    return out


# speedup vs baseline: 5.8129x; 1.0034x over previous
import jax
import jax.numpy as jnp
from jax.experimental import pallas as pl

N = 4096
IN = 256
OUT = 256
HID = 64
HEADS = 2
BI = 1024
HW = 128
H2W = 384


def _dot(a, b):
    return jnp.dot(a, b, preferred_element_type=jnp.float32)


def _leaky(x, slope=0.2):
    return jnp.maximum(x, slope * x)


def _layer_norm(x, g, b, eps=1e-5):
    mu = jnp.mean(x, axis=-1, keepdims=True)
    var = jnp.mean((x - mu) ** 2, axis=-1, keepdims=True)
    return (x - mu) * jax.lax.rsqrt(var + eps) * g + b


def _gelu_exact(x):
    return 0.5 * x * (1.0 + jax.lax.erf(x * (2.0 ** -0.5)))


def _self_loop_weight(g_ref, pid):
    gsub = g_ref[:, pl.ds(pid * BI, BI)].astype(jnp.float32)
    r = jax.lax.broadcasted_iota(jnp.int32, (BI, BI), 0)
    c = jax.lax.broadcasted_iota(jnp.int32, (BI, BI), 1)
    gdiag = jnp.sum(jnp.where(r == c, gsub, 0.0), axis=1, keepdims=True)
    return 1.0 - gdiag


def _h1_kernel(x_ref, w_ref, o_ref):
    h = _dot(x_ref[:, :], w_ref[:, :])
    ones = jnp.ones((h.shape[0], 1), jnp.float32)
    zeros = jnp.zeros((h.shape[0], HW - HID - 1), jnp.float32)
    o_ref[:, :] = jnp.concatenate(
        [h[:, :HID], ones, zeros, h[:, HID:], ones, zeros], axis=-1)


def _gat1_kernel(g_ref, h_ref, ai_ref, aj_ref, b0_ref, w1_ref, h2_ref):
    pid = pl.program_id(0)
    gf = g_ref[:, :].astype(jnp.float32)
    w_self = _self_loop_weight(g_ref, pid)

    cis, cjs, cjbs, hblks = [], [], [], []
    for hd in range(HEADS):
        hh_blk = h_ref[pl.ds(pid * BI, BI), hd * HW:hd * HW + HID]
        hblks.append(h_ref[pl.ds(pid * BI, BI), hd * HW:(hd + 1) * HW])
        cis.append(_dot(hh_blk, ai_ref[hd, :].reshape(HID, 1)))
        cjs.append(_dot(h_ref[:, hd * HW:hd * HW + HID],
                        aj_ref[hd, :].reshape(HID, 1)))
        cjbs.append(_dot(hh_blk, aj_ref[hd, :].reshape(HID, 1)))

    outs = []
    for hd in range(HEADS):
        ci, cj = cis[hd], cjs[hd]
        si = jnp.max(ci)
        shift = _leaky(si + jnp.max(cj))
        sj = shift - si
        eci = jnp.exp(ci - si)
        eci2 = jnp.exp(0.2 * ci - si)
        ecj = jnp.exp(cj - sj).reshape(1, N)
        ecj2 = jnp.exp(0.2 * cj - sj).reshape(1, N)
        p = jnp.maximum(eci * ecj, eci2 * ecj2) * gf
        pd = w_self * jnp.exp(
            _leaky(ci + cjbs[hd]) - shift)
        o_aug = _dot(p, h_ref[:, hd * HW:(hd + 1) * HW]) + pd * hblks[hd]
        outs.append(o_aug[:, :HID] / o_aug[:, HID:HID + 1])
    o = jnp.concatenate(outs, axis=-1) + b0_ref[0, :]
    m1 = jnp.where(o > 0, o, jnp.exp(jnp.minimum(o, 0.0)) - 1.0)
    h2 = _dot(m1, w1_ref[:, :])
    ones = jnp.ones((BI, 1), jnp.float32)
    zeros = jnp.zeros((BI, H2W - OUT - 1), jnp.float32)
    h2_ref[:, :] = jnp.concatenate([h2, ones, zeros], axis=-1)


def _gat2_kernel(g_ref, h2_ref, x_ref, ai_ref, aj_ref, b1_ref,
                 ln1g_ref, ln1b_ref, wl_ref, bl_ref,
                 we1_ref, be1_ref, we2_ref, be2_ref,
                 ln2g_ref, ln2b_ref, wo_ref, bo_ref,
                 lnog_ref, lnob_ref, out_ref):
    pid = pl.program_id(0)
    gf = g_ref[:, :].astype(jnp.float32)
    w_self = _self_loop_weight(g_ref, pid)

    h2_blk = h2_ref[pl.ds(pid * BI, BI), 0:OUT]
    h2aug_blk = h2_ref[pl.ds(pid * BI, BI), :]
    ci = _dot(h2_blk, ai_ref[0, :].reshape(OUT, 1))
    cj = _dot(h2_ref[:, 0:OUT], aj_ref[0, :].reshape(OUT, 1))
    cjb = _dot(h2_blk, aj_ref[0, :].reshape(OUT, 1))
    si = jnp.max(ci)
    shift = _leaky(si + jnp.max(cj))
    sj = shift - si
    eci = jnp.exp(ci - si)
    eci2 = jnp.exp(0.2 * ci - si)
    ecj = jnp.exp(cj - sj).reshape(1, N)
    ecj2 = jnp.exp(0.2 * cj - sj).reshape(1, N)
    p = jnp.maximum(eci * ecj, eci2 * ecj2) * gf
    pd = w_self * jnp.exp(_leaky(ci + cjb) - shift)
    o_aug = _dot(p, h2_ref[:, :]) + pd * h2aug_blk
    o = o_aug[:, :OUT] / o_aug[:, OUT:OUT + 1] + b1_ref[0, :]

    cat = jnp.concatenate([x_ref[:, :], o], axis=-1)
    x = _layer_norm(cat, ln1g_ref[0, :], ln1b_ref[0, :])
    mm = _dot(x, wl_ref[:, :]) + bl_ref[0, :]
    enc = _dot(_gelu_exact(_dot(mm, we1_ref[:, :]) + be1_ref[0, :]),
               we2_ref[:, :]) + be2_ref[0, :]
    out = _layer_norm(mm + enc, ln2g_ref[0, :], ln2b_ref[0, :])
    out = _layer_norm(_gelu_exact(_dot(out, wo_ref[:, :]) + bo_ref[0, :]),
                      lnog_ref[0, :], lnob_ref[0, :])
    out_ref[:, :] = out


def _full(shape):
    nd = len(shape)
    return pl.BlockSpec(shape, lambda i: (0,) * nd)


def kernel(input, graph, W0, ai0, aj0, b0, W1, ai1, aj1, b1, ln1_g, ln1_b,
           Wl, bl, We1, be1, We2, be2, ln2_g, ln2_b, Wo, bo, lno_g, lno_b):
    nblk = N // BI
    row2 = lambda v: v.reshape(1, -1)

    h1 = pl.pallas_call(
        _h1_kernel,
        grid=(nblk,),
        in_specs=[pl.BlockSpec((BI, IN), lambda i: (i, 0)),
                  _full((IN, HEADS * HID))],
        out_specs=pl.BlockSpec((BI, HEADS * HW), lambda i: (i, 0)),
        out_shape=jax.ShapeDtypeStruct((N, HEADS * HW), jnp.float32),
    )(input, W0)

    h2 = pl.pallas_call(
        _gat1_kernel,
        grid=(nblk,),
        in_specs=[
            pl.BlockSpec((BI, N), lambda i: (i, 0)),
            _full((N, HEADS * HW)),
            _full((HEADS, HID)),
            _full((HEADS, HID)),
            _full((1, HEADS * HID)),
            _full((HEADS * HID, OUT)),
        ],
        out_specs=pl.BlockSpec((BI, H2W), lambda i: (i, 0)),
        out_shape=jax.ShapeDtypeStruct((N, H2W), jnp.float32),
    )(graph, h1, ai0, aj0, row2(b0), W1)

    out = pl.pallas_call(
        _gat2_kernel,
        grid=(nblk,),
        in_specs=[
            pl.BlockSpec((BI, N), lambda i: (i, 0)),
            _full((N, H2W)),
            pl.BlockSpec((BI, IN), lambda i: (i, 0)),
            _full((1, OUT)),
            _full((1, OUT)),
            _full((1, OUT)),
            _full((1, IN + OUT)),
            _full((1, IN + OUT)),
            _full((IN + OUT, OUT)),
            _full((1, OUT)),
            _full((OUT, OUT)),
            _full((1, OUT)),
            _full((OUT, OUT)),
            _full((1, OUT)),
            _full((1, OUT)),
            _full((1, OUT)),
            _full((OUT, OUT)),
            _full((1, OUT)),
            _full((1, OUT)),
            _full((1, OUT)),
        ],
        out_specs=pl.BlockSpec((BI, OUT), lambda i: (i, 0)),
        out_shape=jax.ShapeDtypeStruct((N, OUT), jnp.float32),
    )(graph, h2, input, ai1, aj1, row2(b1), row2(ln1_g), row2(ln1_b), Wl,
      row2(bl), We1, row2(be1), We2, row2(be2), row2(ln2_g), row2(ln2_b),
      Wo, row2(bo), row2(lno_g), row2(lno_b))
    return out
